# Initial kernel scaffold; baseline (speedup 1.0000x reference)
import jax
import jax.numpy as jnp
from jax.experimental import pallas as pl


def kernel(batch_node_pred, batch_node_input, batch_edge_input, total_rainfall, batch, edge_index, boundary_nodes_mask, inflow_edges_mask, outflow_edges_mask, node_mean, node_std, edge_mean, edge_std):
    raise NotImplementedError("write your pallas kernel here")


# speedup vs baseline: 183.0983x; 183.0983x over previous
import functools

import jax
import jax.numpy as jnp
from jax import lax
from jax.experimental import pallas as pl
from jax.experimental.pallas import tpu as pltpu
from jax.experimental.pallas import tpu_sc as plsc

N = 100000
E = 6400000
B = 16
DT = 30.0

NC = 2
NS = 16
NW = NC * NS
L = 16

EW = E // NW
CE = 2000
NCH = EW // CE

NP = NW * 3136
NBV = 3136 // L


def _sc_body(src_h, dst_h, flow_h, inm_h, outm_h, nin_h, npr_h, bnd_h,
             batch_h, scal_h, zer_h, out_h,
             tbl, srcb, dstb, flwb, inmb, outmb, ninb, nprb, bndb,
             accb, scalb):
    wid = lax.axis_index("s") * NC + lax.axis_index("c")

    pltpu.sync_copy(batch_h, tbl)
    pltpu.sync_copy(scal_h, scalb)
    pltpu.sync_copy(zer_h, accb)

    estd = scalb[pl.ds(0, L)]
    emean = scalb[pl.ds(L, L)]
    nstd = scalb[pl.ds(2 * L, L)]
    lane16 = lax.iota(jnp.int32, L) * B

    nbase = wid * 3136
    pltpu.sync_copy(nin_h.at[pl.ds(nbase, 3136)], ninb)
    pltpu.sync_copy(npr_h.at[pl.ds(nbase, 3136)], nprb)
    pltpu.sync_copy(bnd_h.at[pl.ds(nbase, 3136)], bndb)

    def nvec_body(iv, carry):
        sl = pl.ds(iv * L, L)
        bv = tbl[pl.ds(nbase + iv * L, L)]
        v = (nprb[sl] - ninb[sl]) * nstd * (1.0 - bndb[sl])
        plsc.addupdate_scatter(accb, [lane16 + bv], v)
        return carry

    lax.fori_loop(0, NBV, nvec_body, 0)

    def chunk_body(k, carry):
        base = wid * EW + k * CE
        pltpu.sync_copy(src_h.at[pl.ds(base, CE)], srcb)
        pltpu.sync_copy(dst_h.at[pl.ds(base, CE)], dstb)
        pltpu.sync_copy(flow_h.at[pl.ds(base, CE)], flwb)
        pltpu.sync_copy(inm_h.at[pl.ds(base, CE)], inmb)
        pltpu.sync_copy(outm_h.at[pl.ds(base, CE)], outmb)

        def vec_body(iv, c2):
            sl = pl.ds(iv * L, L)
            flw = flwb[sl] * estd + emean
            g1 = plsc.load_gather(tbl, [srcb[sl]])
            g2 = plsc.load_gather(tbl, [dstb[sl]])
            plsc.addupdate_scatter(accb, [lane16 + g1], flw * inmb[sl] * (-DT))
            plsc.addupdate_scatter(accb, [lane16 + g2], flw * outmb[sl] * DT)
            return c2

        lax.fori_loop(0, CE // L, vec_body, 0)
        return carry

    lax.fori_loop(0, NCH, chunk_body, 0)

    pltpu.sync_copy(accb, out_h.at[wid])


def _tc_body(parts_ref, rain_ref, o_ref):
    s = jnp.sum(parts_ref[...], axis=0, keepdims=True)
    err = s - rain_ref[...]
    o_ref[...] = jnp.sum(jnp.abs(err), axis=1, keepdims=True) * (1.0 / B)


def kernel(batch_node_pred, batch_node_input, batch_edge_input, total_rainfall,
           batch, edge_index, boundary_nodes_mask, inflow_edges_mask,
           outflow_edges_mask, node_mean, node_std, edge_mean, edge_std):
    f32 = jnp.float32
    src = edge_index[0].astype(jnp.int32)
    dst = edge_index[1].astype(jnp.int32)
    flow = batch_edge_input[:, 0]
    inm = inflow_edges_mask.astype(f32)
    outm = outflow_edges_mask.astype(f32)
    pad = NP - N
    nin = jnp.pad(batch_node_input[:, 0], (0, pad))
    npr = jnp.pad(batch_node_pred[:, 0], (0, pad))
    bnd = jnp.pad(boundary_nodes_mask.astype(f32), (0, pad),
                  constant_values=1.0)
    batchp = jnp.pad(batch.astype(jnp.int32), (0, pad))
    scal = jnp.concatenate([jnp.full((L,), edge_std, f32),
                            jnp.full((L,), edge_mean, f32),
                            jnp.full((L,), node_std, f32)])
    zer = jnp.zeros((L * B,), f32)

    mesh = plsc.VectorSubcoreMesh(core_axis_name="c", subcore_axis_name="s",
                                  num_cores=NC, num_subcores=NS)
    parts = pl.kernel(
        _sc_body,
        out_type=jax.ShapeDtypeStruct((NW, L * B), f32),
        mesh=mesh,
        compiler_params=pltpu.CompilerParams(needs_layout_passes=False),
        scratch_types=[
            pltpu.VMEM((NP,), jnp.int32),
            pltpu.VMEM((CE,), jnp.int32),
            pltpu.VMEM((CE,), jnp.int32),
            pltpu.VMEM((CE,), f32),
            pltpu.VMEM((CE,), f32),
            pltpu.VMEM((CE,), f32),
            pltpu.VMEM((3136,), f32),
            pltpu.VMEM((3136,), f32),
            pltpu.VMEM((3136,), f32),
            pltpu.VMEM((L * B,), f32),
            pltpu.VMEM((3 * L,), f32),
        ],
    )(src, dst, flow, inm, outm, nin, npr, bnd, batchp, scal, zer)

    loss = pl.pallas_call(
        _tc_body,
        out_shape=jax.ShapeDtypeStruct((1, 1), f32),
    )(parts.reshape(NW * L, B), total_rainfall.reshape(1, B))
    return loss[0, 0]


# speedup vs baseline: 320.4078x; 1.7499x over previous
import functools

import jax
import jax.numpy as jnp
from jax import lax
from jax.experimental import pallas as pl
from jax.experimental.pallas import tpu as pltpu
from jax.experimental.pallas import tpu_sc as plsc

N = 100000
E = 6400000
B = 16
DT = 30.0

NC = 2
NS = 16
NW = NC * NS
L = 16

EW = E // NW
CE = 2000
NCH = EW // CE

NP = NW * 3136
NBV = 3136 // L


def _sc_body(src_h, dst_h, flow_h, inm_h, outm_h, nin_h, npr_h, bnd_h,
             batch_h, scal_h, zer_h, out_h,
             tbl, srcbA, dstbA, flwbA, inmbA, outmbA,
             srcbB, dstbB, flwbB, inmbB, outmbB,
             ninb, nprb, bndb, accb, scalb, semA, semB):
    wid = lax.axis_index("s") * NC + lax.axis_index("c")

    edge_hbm = (src_h, dst_h, flow_h, inm_h, outm_h)
    setA = (srcbA, dstbA, flwbA, inmbA, outmbA)
    setB = (srcbB, dstbB, flwbB, inmbB, outmbB)

    def issue(k, bufs, sem):
        base = pl.multiple_of(wid * EW + k * CE, 8)
        for h, b in zip(edge_hbm, bufs):
            pltpu.async_copy(h.at[pl.ds(base, CE)], b, sem)

    def drain(bufs, sem):
        for h, b in zip(edge_hbm, bufs):
            pltpu.make_async_copy(h.at[pl.ds(0, CE)], b, sem).wait()

    issue(0, setA, semA)
    issue(1, setB, semB)

    pltpu.sync_copy(batch_h, tbl)
    pltpu.sync_copy(scal_h, scalb)
    pltpu.sync_copy(zer_h, accb)

    estd = scalb[pl.ds(0, L)]
    emean = scalb[pl.ds(L, L)]
    nstd = scalb[pl.ds(2 * L, L)]
    lane16 = lax.iota(jnp.int32, L) * B

    nbase = wid * 3136
    pltpu.sync_copy(nin_h.at[pl.ds(nbase, 3136)], ninb)
    pltpu.sync_copy(npr_h.at[pl.ds(nbase, 3136)], nprb)
    pltpu.sync_copy(bnd_h.at[pl.ds(nbase, 3136)], bndb)

    def nvec_body(iv, carry):
        sl = pl.ds(iv * L, L)
        bv = tbl[pl.ds(nbase + iv * L, L)]
        v = (nprb[sl] - ninb[sl]) * nstd * (1.0 - bndb[sl])
        plsc.addupdate_scatter(accb, [lane16 + bv], v)
        return carry

    lax.fori_loop(0, NBV, nvec_body, 0, unroll=4)

    def consume(bufs):
        srcb, dstb, flwb, inmb, outmb = bufs

        def vec_body(iv, c2):
            sl = pl.ds(iv * L, L)
            flw = flwb[sl] * estd + emean
            g1 = plsc.load_gather(tbl, [srcb[sl]])
            g2 = plsc.load_gather(tbl, [dstb[sl]])
            plsc.addupdate_scatter(accb, [lane16 + g1], flw * inmb[sl] * (-DT))
            plsc.addupdate_scatter(accb, [lane16 + g2], flw * outmb[sl] * DT)
            return c2

        lax.fori_loop(0, CE // L, vec_body, 0, unroll=5)

    def pair_body(j, carry):
        k0 = 2 * j
        drain(setA, semA)
        consume(setA)

        @pl.when(k0 + 2 < NCH)
        def _():
            issue(k0 + 2, setA, semA)

        drain(setB, semB)
        consume(setB)

        @pl.when(k0 + 3 < NCH)
        def _():
            issue(k0 + 3, setB, semB)

        return carry

    lax.fori_loop(0, NCH // 2, pair_body, 0)

    pltpu.sync_copy(accb, out_h.at[wid])


def _tc_body(parts_ref, rain_ref, o_ref):
    s = jnp.sum(parts_ref[...], axis=0, keepdims=True)
    err = s - rain_ref[...]
    o_ref[...] = jnp.sum(jnp.abs(err), axis=1, keepdims=True) * (1.0 / B)


def kernel(batch_node_pred, batch_node_input, batch_edge_input, total_rainfall,
           batch, edge_index, boundary_nodes_mask, inflow_edges_mask,
           outflow_edges_mask, node_mean, node_std, edge_mean, edge_std):
    f32 = jnp.float32
    src = edge_index[0].astype(jnp.int32)
    dst = edge_index[1].astype(jnp.int32)
    flow = batch_edge_input[:, 0]
    inm = inflow_edges_mask.astype(f32)
    outm = outflow_edges_mask.astype(f32)
    pad = NP - N
    nin = jnp.pad(batch_node_input[:, 0], (0, pad))
    npr = jnp.pad(batch_node_pred[:, 0], (0, pad))
    bnd = jnp.pad(boundary_nodes_mask.astype(f32), (0, pad),
                  constant_values=1.0)
    batchp = jnp.pad(batch.astype(jnp.int32), (0, pad))
    scal = jnp.concatenate([jnp.full((L,), edge_std, f32),
                            jnp.full((L,), edge_mean, f32),
                            jnp.full((L,), node_std, f32)])
    zer = jnp.zeros((L * B,), f32)

    mesh = plsc.VectorSubcoreMesh(core_axis_name="c", subcore_axis_name="s",
                                  num_cores=NC, num_subcores=NS)
    parts = pl.kernel(
        _sc_body,
        out_type=jax.ShapeDtypeStruct((NW, L * B), f32),
        mesh=mesh,
        compiler_params=pltpu.CompilerParams(needs_layout_passes=False),
        scratch_types=(
            [pltpu.VMEM((NP,), jnp.int32)]
            + 2 * [pltpu.VMEM((CE,), jnp.int32),
                   pltpu.VMEM((CE,), jnp.int32),
                   pltpu.VMEM((CE,), f32),
                   pltpu.VMEM((CE,), f32),
                   pltpu.VMEM((CE,), f32)]
            + [pltpu.VMEM((3136,), f32),
               pltpu.VMEM((3136,), f32),
               pltpu.VMEM((3136,), f32),
               pltpu.VMEM((L * B,), f32),
               pltpu.VMEM((3 * L,), f32),
               pltpu.SemaphoreType.DMA,
               pltpu.SemaphoreType.DMA]
        ),
    )(src, dst, flow, inm, outm, nin, npr, bnd, batchp, scal, zer)

    loss = pl.pallas_call(
        _tc_body,
        out_shape=jax.ShapeDtypeStruct((1, 1), f32),
    )(parts.reshape(NW * L, B), total_rainfall.reshape(1, B))
    return loss[0, 0]
